# baseline (speedup 1.0000x reference)
import functools

import jax
import jax.numpy as jnp
from jax import lax
from jax.experimental import pallas as pl
from jax.experimental.pallas import tpu as pltpu
from jax.experimental.pallas import tpu_sc as plsc

N = 10000
E = 320000
F = 128
H1, D1 = 8, 8
C = 40
CP = 48
NP = 10112
EP = 327680
ROWS = EP // 128
NT = 16
NC = 2

f32 = jnp.float32
i32 = jnp.int32


def _iota16():
    return lax.iota(i32, 16)


def _vperm(x, idx):
    dnums = lax.GatherDimensionNumbers(
        offset_dims=(), collapsed_slice_dims=(0,), start_index_map=(0,))
    return lax.gather(x, idx[:, None], dnums, (1,),
                      mode=lax.GatherScatterMode.PROMISE_IN_BOUNDS)


def _tc1_body(x_ref, w_ref, al_ref, ar_ref, h_ref, el_ref, er_ref):
    h = jnp.dot(x_ref[...], w_ref[...], preferred_element_type=f32)
    h_ref[0] = h[:, :32]
    h_ref[1] = h[:, 32:]
    el_ref[...] = jnp.dot(h, al_ref[...], preferred_element_type=f32)
    er_ref[...] = jnp.dot(h, ar_ref[...], preferred_element_type=f32)


def _tc1(x, w1, albk, arbk):
    bn = 1000
    return pl.pallas_call(
        _tc1_body,
        grid=(N // bn,),
        in_specs=[
            pl.BlockSpec((bn, F), lambda i: (i, 0)),
            pl.BlockSpec((F, 64), lambda i: (0, 0)),
            pl.BlockSpec((64, 8), lambda i: (0, 0)),
            pl.BlockSpec((64, 8), lambda i: (0, 0)),
        ],
        out_specs=[
            pl.BlockSpec((2, bn, 32), lambda i: (0, i, 0)),
            pl.BlockSpec((bn, 8), lambda i: (i, 0)),
            pl.BlockSpec((bn, 8), lambda i: (i, 0)),
        ],
        out_shape=[
            jax.ShapeDtypeStruct((2, N, 32), f32),
            jax.ShapeDtypeStruct((N, 8), f32),
            jax.ShapeDtypeStruct((N, 8), f32),
        ],
    )(x, w1, albk, arbk)


def _tc2_body(o1_ref, w2_ref, b1_ref, a2l_ref, a2r_ref, h2_ref, el_ref, er_ref):
    v = o1_ref[...] + b1_ref[...]
    y = jnp.where(v > 0, v, jnp.exp(v) - 1.0)
    h2 = jnp.dot(y, w2_ref[...], preferred_element_type=f32)
    h2_ref[...] = h2
    el_ref[...] = jnp.dot(h2, a2l_ref[...], preferred_element_type=f32)
    er_ref[...] = jnp.dot(h2, a2r_ref[...], preferred_element_type=f32)


def _tc2(o1, w2p, b1r, a2l, a2r):
    bn = 1000
    return pl.pallas_call(
        _tc2_body,
        grid=(N // bn,),
        in_specs=[
            pl.BlockSpec((bn, 64), lambda i: (i, 0)),
            pl.BlockSpec((64, CP), lambda i: (0, 0)),
            pl.BlockSpec((1, 64), lambda i: (0, 0)),
            pl.BlockSpec((CP, 1), lambda i: (0, 0)),
            pl.BlockSpec((CP, 1), lambda i: (0, 0)),
        ],
        out_specs=[
            pl.BlockSpec((bn, CP), lambda i: (i, 0)),
            pl.BlockSpec((bn, 1), lambda i: (i, 0)),
            pl.BlockSpec((bn, 1), lambda i: (i, 0)),
        ],
        out_shape=[
            jax.ShapeDtypeStruct((N, CP), f32),
            jax.ShapeDtypeStruct((N, 1), f32),
            jax.ShapeDtypeStruct((N, 1), f32),
        ],
    )(o1, w2p, b1r, a2l, a2r)


def _tc3_body(p0_ref, p1_ref, b2_ref, out_ref):
    z = p0_ref[...] + p1_ref[...] + b2_ref[...]
    col = lax.broadcasted_iota(i32, z.shape, 1)
    mask = col < C
    zm = jnp.where(mask, z, -1e30)
    m = jnp.max(zm, axis=1, keepdims=True)
    ex = jnp.where(mask, jnp.exp(z - m), 0.0)
    s = jnp.sum(ex, axis=1, keepdims=True)
    out_ref[...] = z - m - jnp.log(s)


def _tc3(p0, p1, b2r):
    bn = 1000
    return pl.pallas_call(
        _tc3_body,
        grid=(N // bn,),
        in_specs=[
            pl.BlockSpec((bn, CP), lambda i: (i, 0)),
            pl.BlockSpec((bn, CP), lambda i: (i, 0)),
            pl.BlockSpec((1, CP), lambda i: (0, 0)),
        ],
        out_specs=pl.BlockSpec((bn, CP), lambda i: (i, 0)),
        out_shape=jax.ShapeDtypeStruct((N, CP), f32),
    )(p0, p1, b2r)


def _sc1_body(elc, erc, h1f, src2, dst2, z16, z32, outg,
              elv, erv, sidx, didx, eebuf, scatbuf, dbuf, hbufs, msgbuf,
              sem, dtab, outs):
    cid = lax.axis_index("c")
    sid = lax.axis_index("s")
    io = _iota16()
    m4 = io < 4

    pltpu.sync_copy(elc.at[cid], elv)
    pltpu.sync_copy(erc.at[cid], erv)
    nz0 = sid * 632
    pltpu.sync_copy(z16, dtab.at[pl.ds(nz0, 632)])
    pltpu.sync_copy(z32, outs.at[pl.ds(nz0, 632)])
    plsc.subcore_barrier()

    r0 = sid * (ROWS // NT)

    def group_a(g, _):
        rg = r0 + g * 4
        pltpu.sync_copy(src2.at[pl.ds(rg, 4)], sidx)
        pltpu.sync_copy(dst2.at[pl.ds(rg, 4)], didx)

        def row_a(m, _):
            for h in range(4):
                for s in range(8):
                    sv = sidx[m, pl.ds(s * 16, 16)]
                    dv = didx[m, pl.ds(s * 16, 16)]
                    a = plsc.load_gather(elv, [sv + h * N])
                    b = plsc.load_gather(erv, [dv + h * N])
                    e = a + b
                    e = jnp.maximum(e, 0.2 * e)
                    eebuf[pl.ds(m * 512 + h * 128 + s * 16, 16)] = jnp.exp(e)
            cb = jnp.where(m4, m * 512 + io * 128, 0)

            def tr_a(eg, ev):
                for q in range(8):
                    erow = plsc.load_gather(eebuf, [ev + q])
                    scatbuf[eg * 8 + q] = jnp.where(m4, erow, 0.0)
                return ev + 8

            lax.fori_loop(0, 16, tr_a, cb)
            pltpu.sync_copy(scatbuf, dtab.at[didx.at[m]], add=True)
            return 0

        lax.fori_loop(0, 4, row_a, 0)
        return 0

    lax.fori_loop(0, (ROWS // NT) // 4, group_a, 0)
    plsc.subcore_barrier()

    def inv_c(c, _):
        base = nz0 + c * 8
        pltpu.sync_copy(dtab.at[pl.ds(base, 8)], scatbuf.at[pl.ds(0, 8)])
        for r in range(8):
            scatbuf[r] = 1.0 / (scatbuf[r] + 1e-9)
        pltpu.sync_copy(scatbuf.at[pl.ds(0, 8)], dtab.at[pl.ds(base, 8)])
        return 0

    lax.fori_loop(0, 632 // 8, inv_c, 0)
    plsc.subcore_barrier()

    coff = cid * N

    def group_c(g, _):
        rg = r0 + g * 4
        pltpu.sync_copy(src2.at[pl.ds(rg, 4)], sidx)
        pltpu.sync_copy(dst2.at[pl.ds(rg, 4)], didx)

        def row_ee(m, _):
            for h in range(4):
                for s in range(8):
                    sv = sidx[m, pl.ds(s * 16, 16)]
                    dv = didx[m, pl.ds(s * 16, 16)]
                    a = plsc.load_gather(elv, [sv + h * N])
                    b = plsc.load_gather(erv, [dv + h * N])
                    e = a + b
                    e = jnp.maximum(e, 0.2 * e)
                    eebuf[pl.ds(m * 512 + h * 128 + s * 16, 16)] = jnp.exp(e)
            return 0

        lax.fori_loop(0, 4, row_ee, 0)
        for mm in range(4):
            for s in range(8):
                sidx[mm, pl.ds(s * 16, 16)] = sidx[mm, pl.ds(s * 16, 16)] + coff

        def pair_c(pp, _):
            d0 = pltpu.async_copy(h1f.at[sidx.at[2 * pp]],
                                  hbufs.at[pl.ds(0, 128)], sem)
            d1 = pltpu.async_copy(h1f.at[sidx.at[2 * pp + 1]],
                                  hbufs.at[pl.ds(128, 128)], sem)
            d0.wait()
            d1.wait()

            def row_c(m, _):
                r = 2 * pp + m
                pltpu.sync_copy(dtab.at[didx.at[r]], dbuf)
                hb = m * 128
                cb = jnp.where(m4, r * 512 + io * 128, 0)
                pats = [jnp.where(io < 8, 2 * j, 2 * j + 1) for j in range(2)]

                def edge_c(eg, ev):
                    for q in range(8):
                        e = eg * 8 + q
                        erow = plsc.load_gather(eebuf, [ev + q])
                        alpha = erow * dbuf[e]
                        for j in range(2):
                            aexp = _vperm(alpha, pats[j])
                            hv = hbufs[hb + e, pl.ds(j * 16, 16)]
                            msgbuf[e, pl.ds(j * 16, 16)] = hv * aexp
                    return ev + 8

                lax.fori_loop(0, 16, edge_c, cb)
                pltpu.sync_copy(msgbuf, outs.at[didx.at[r]], add=True)
                return 0

            lax.fori_loop(0, 2, row_c, 0)
            return 0

        lax.fori_loop(0, 2, pair_c, 0)
        return 0

    lax.fori_loop(0, (ROWS // NT) // 4, group_c, 0)
    plsc.subcore_barrier()
    pltpu.sync_copy(outs.at[pl.ds(nz0, 632)], outg.at[cid, pl.ds(nz0, 632)])


def _sc1(elc, erc, h1f, src2, dst2, z16, z32):
    mesh = plsc.VectorSubcoreMesh(core_axis_name="c", subcore_axis_name="s")
    return pl.kernel(
        _sc1_body,
        out_type=jax.ShapeDtypeStruct((2, NP, 32), f32),
        mesh=mesh,
        compiler_params=pltpu.CompilerParams(needs_layout_passes=False, use_tc_tiling_on_sc=False),
        scratch_types=[
            pltpu.VMEM((4 * N,), f32),
            pltpu.VMEM((4 * N,), f32),
            pltpu.VMEM((4, 128), i32),
            pltpu.VMEM((4, 128), i32),
            pltpu.VMEM((4 * 512,), f32),
            pltpu.VMEM((128, 16), f32),
            pltpu.VMEM((128, 16), f32),
            pltpu.VMEM((256, 32), f32),
            pltpu.VMEM((128, 32), f32),
            pltpu.SemaphoreType.DMA,
            pltpu.VMEM_SHARED((NP, 16), f32),
            pltpu.VMEM_SHARED((NP, 32), f32),
        ],
    )(elc, erc, h1f, src2, dst2, z16, z32)


def _sc2_body(el2, er2, h2g, src2, dst2, z16, z48, outg,
              elv, erv, sidx, didx, eebuf, scatbuf, dbuf, hbufs, msgbuf,
              sem, ees, dtab, outs):
    cid = lax.axis_index("c")
    sid = lax.axis_index("s")
    io = _iota16()
    m1 = io < 1
    zvec = jnp.zeros((16,), i32)

    pltpu.sync_copy(el2, elv)
    pltpu.sync_copy(er2, erv)
    nz0 = sid * 632
    pltpu.sync_copy(z16, dtab.at[pl.ds(nz0, 632)])
    pltpu.sync_copy(z48, outs.at[pl.ds(nz0, 632)])
    plsc.subcore_barrier()

    r0 = sid * (ROWS // NT)
    half = ROWS // 2
    save = (sid * (ROWS // NT)) // half == cid

    def group_a(g, _):
        rg = r0 + g * 8
        pltpu.sync_copy(src2.at[pl.ds(rg, 8)], sidx)
        pltpu.sync_copy(dst2.at[pl.ds(rg, 8)], didx)

        def row_a(m, _):
            def sgrp_a(sg, _):
                sv = sidx[m, pl.ds(sg * 16, 16)]
                dv = didx[m, pl.ds(sg * 16, 16)]
                a = plsc.load_gather(elv, [sv])
                b = plsc.load_gather(erv, [dv])
                e = a + b
                e = jnp.maximum(e, 0.2 * e)
                ee = jnp.exp(e)
                eebuf[pl.ds(m * 128 + sg * 16, 16)] = ee
                for l in range(16):
                    asp = _vperm(ee, jnp.full((16,), l, i32))
                    scatbuf[sg * 16 + l] = jnp.where(m1, asp, 0.0)
                return 0

            lax.fori_loop(0, 8, sgrp_a, 0)
            pltpu.sync_copy(scatbuf, dtab.at[didx.at[m]], add=True)
            return 0

        lax.fori_loop(0, 8, row_a, 0)

        @pl.when(save)
        def _():
            pltpu.sync_copy(eebuf, ees.at[pl.ds((rg - cid * half) * 128, 1024)])

        return 0

    lax.fori_loop(0, (ROWS // NT) // 8, group_a, 0)
    plsc.subcore_barrier()

    def inv_c(c, _):
        base = nz0 + c * 8
        pltpu.sync_copy(dtab.at[pl.ds(base, 8)], scatbuf.at[pl.ds(0, 8)])
        for r in range(8):
            scatbuf[r] = 1.0 / (scatbuf[r] + 1e-9)
        pltpu.sync_copy(scatbuf.at[pl.ds(0, 8)], dtab.at[pl.ds(base, 8)])
        return 0

    lax.fori_loop(0, 632 // 8, inv_c, 0)
    plsc.subcore_barrier()

    rpw = half // NT
    rc0 = sid * rpw

    def group_c(g, _):
        rl = rc0 + g * 8
        rg = cid * half + rl
        pltpu.sync_copy(src2.at[pl.ds(rg, 8)], sidx)
        pltpu.sync_copy(dst2.at[pl.ds(rg, 8)], didx)
        pltpu.sync_copy(ees.at[pl.ds(rl * 128, 1024)], eebuf)

        def pair_c(pp, _):
            descs = [
                pltpu.async_copy(h2g.at[sidx.at[4 * pp + mm]],
                                 hbufs.at[pl.ds(mm * 128, 128)], sem)
                for mm in range(4)
            ]
            for d in descs:
                d.wait()

            def row_c(m, _):
                r = 4 * pp + m
                pltpu.sync_copy(dtab.at[didx.at[r]], dbuf)
                hb = m * 128

                def sgrp_c(sg, _):
                    dcol = plsc.load_gather(dbuf, [io + sg * 16, zvec])
                    al = eebuf[pl.ds(r * 128 + sg * 16, 16)] * dcol
                    for l in range(16):
                        asp = _vperm(al, jnp.full((16,), l, i32))
                        for j in range(3):
                            hv = hbufs[hb + sg * 16 + l, pl.ds(j * 16, 16)]
                            msgbuf[sg * 16 + l, pl.ds(j * 16, 16)] = hv * asp
                    return 0

                lax.fori_loop(0, 8, sgrp_c, 0)
                pltpu.sync_copy(msgbuf, outs.at[didx.at[r]], add=True)
                return 0

            lax.fori_loop(0, 4, row_c, 0)
            return 0

        lax.fori_loop(0, 2, pair_c, 0)
        return 0

    lax.fori_loop(0, rpw // 8, group_c, 0)
    plsc.subcore_barrier()
    pltpu.sync_copy(outs.at[pl.ds(nz0, 632)], outg.at[cid, pl.ds(nz0, 632)])


def _sc2(el2, er2, h2g, src2, dst2, z16, z48):
    mesh = plsc.VectorSubcoreMesh(core_axis_name="c", subcore_axis_name="s")
    return pl.kernel(
        _sc2_body,
        out_type=jax.ShapeDtypeStruct((2, NP, CP), f32),
        mesh=mesh,
        compiler_params=pltpu.CompilerParams(needs_layout_passes=False, use_tc_tiling_on_sc=False),
        scratch_types=[
            pltpu.VMEM((N,), f32),
            pltpu.VMEM((N,), f32),
            pltpu.VMEM((8, 128), i32),
            pltpu.VMEM((8, 128), i32),
            pltpu.VMEM((8 * 128,), f32),
            pltpu.VMEM((128, 16), f32),
            pltpu.VMEM((128, 16), f32),
            pltpu.VMEM((512, CP), f32),
            pltpu.VMEM((128, CP), f32),
            pltpu.SemaphoreType.DMA,
            pltpu.VMEM_SHARED((ROWS // 2 * 128,), f32),
            pltpu.VMEM_SHARED((NP, 16), f32),
            pltpu.VMEM_SHARED((NP, CP), f32),
        ],
    )(el2, er2, h2g, src2, dst2, z16, z48)


@jax.jit
def kernel(inputs, edge_index, W1, b1, al1, ar1, W2, b2, al2, ar2):
    src = edge_index[0].astype(i32)
    dst = edge_index[1].astype(i32)

    eye = jnp.repeat(jnp.eye(H1, dtype=f32), D1, axis=0)
    albk = eye * al1.reshape(H1 * D1, 1)
    arbk = eye * ar1.reshape(H1 * D1, 1)

    h1g, el, er = _tc1(inputs, W1, albk, arbk)

    npad = EP - E
    srcp = jnp.concatenate([src, jnp.zeros((npad,), i32)]).reshape(ROWS, 128)
    dstp = jnp.concatenate([dst, jnp.full((npad,), N, i32)]).reshape(ROWS, 128)

    elc = el.T.reshape(2, 4 * N)
    erc = er.T.reshape(2, 4 * N)
    h1f = h1g.reshape(2 * N, 32)
    z16 = jnp.zeros((632, 16), f32)
    z32 = jnp.zeros((632, 32), f32)
    z48 = jnp.zeros((632, CP), f32)

    o1g = _sc1(elc, erc, h1f, srcp, dstp, z16, z32)
    o1 = jnp.concatenate([o1g[0, :N], o1g[1, :N]], axis=1)

    w2p = jnp.pad(W2, ((0, 0), (0, CP - C)))
    a2l = jnp.pad(al2.reshape(C, 1), ((0, CP - C), (0, 0)))
    a2r = jnp.pad(ar2.reshape(C, 1), ((0, CP - C), (0, 0)))
    h2g, el2, er2 = _tc2(o1, w2p, b1.reshape(1, 64), a2l, a2r)

    o2g = _sc2(el2.reshape(N), er2.reshape(N), h2g, srcp, dstp, z16, z48)

    b2r = jnp.pad(b2, (0, CP - C)).reshape(1, CP)
    out = _tc3(o2g[0, :N], o2g[1, :N], b2r)
    return out[:, :C]


# speedup vs baseline: 1.2474x; 1.2474x over previous
import functools

import jax
import jax.numpy as jnp
from jax import lax
from jax.experimental import pallas as pl
from jax.experimental.pallas import tpu as pltpu
from jax.experimental.pallas import tpu_sc as plsc

N = 10000
E = 320000
F = 128
H1, D1 = 8, 8
C = 40
CP = 48
NP = 10112
EP = 327680
ROWS = EP // 128
NT = 16
NC = 2

f32 = jnp.float32
i32 = jnp.int32


def _iota16():
    return lax.iota(i32, 16)


def _vperm(x, idx):
    dnums = lax.GatherDimensionNumbers(
        offset_dims=(), collapsed_slice_dims=(0,), start_index_map=(0,))
    return lax.gather(x, idx[:, None], dnums, (1,),
                      mode=lax.GatherScatterMode.PROMISE_IN_BOUNDS)


def _tc1_body(x_ref, w_ref, al_ref, ar_ref, h_ref, el_ref, er_ref):
    h = jnp.dot(x_ref[...], w_ref[...], preferred_element_type=f32)
    h_ref[0] = h[:, :32]
    h_ref[1] = h[:, 32:]
    el_ref[...] = jnp.dot(h, al_ref[...], preferred_element_type=f32)
    er_ref[...] = jnp.dot(h, ar_ref[...], preferred_element_type=f32)


def _tc1(x, w1, albk, arbk):
    bn = 1000
    return pl.pallas_call(
        _tc1_body,
        grid=(N // bn,),
        in_specs=[
            pl.BlockSpec((bn, F), lambda i: (i, 0)),
            pl.BlockSpec((F, 64), lambda i: (0, 0)),
            pl.BlockSpec((64, 8), lambda i: (0, 0)),
            pl.BlockSpec((64, 8), lambda i: (0, 0)),
        ],
        out_specs=[
            pl.BlockSpec((2, bn, 32), lambda i: (0, i, 0)),
            pl.BlockSpec((bn, 8), lambda i: (i, 0)),
            pl.BlockSpec((bn, 8), lambda i: (i, 0)),
        ],
        out_shape=[
            jax.ShapeDtypeStruct((2, N, 32), f32),
            jax.ShapeDtypeStruct((N, 8), f32),
            jax.ShapeDtypeStruct((N, 8), f32),
        ],
    )(x, w1, albk, arbk)


def _tc2_body(o1_ref, w2_ref, b1_ref, a2l_ref, a2r_ref, h2_ref, el_ref, er_ref):
    v = o1_ref[...] + b1_ref[...]
    y = jnp.where(v > 0, v, jnp.exp(v) - 1.0)
    h2 = jnp.dot(y, w2_ref[...], preferred_element_type=f32)
    h2_ref[...] = h2
    el_ref[...] = jnp.dot(h2, a2l_ref[...], preferred_element_type=f32)
    er_ref[...] = jnp.dot(h2, a2r_ref[...], preferred_element_type=f32)


def _tc2(o1, w2p, b1r, a2l, a2r):
    bn = 1000
    return pl.pallas_call(
        _tc2_body,
        grid=(N // bn,),
        in_specs=[
            pl.BlockSpec((bn, 64), lambda i: (i, 0)),
            pl.BlockSpec((64, CP), lambda i: (0, 0)),
            pl.BlockSpec((1, 64), lambda i: (0, 0)),
            pl.BlockSpec((CP, 1), lambda i: (0, 0)),
            pl.BlockSpec((CP, 1), lambda i: (0, 0)),
        ],
        out_specs=[
            pl.BlockSpec((bn, CP), lambda i: (i, 0)),
            pl.BlockSpec((bn, 1), lambda i: (i, 0)),
            pl.BlockSpec((bn, 1), lambda i: (i, 0)),
        ],
        out_shape=[
            jax.ShapeDtypeStruct((N, CP), f32),
            jax.ShapeDtypeStruct((N, 1), f32),
            jax.ShapeDtypeStruct((N, 1), f32),
        ],
    )(o1, w2p, b1r, a2l, a2r)


def _tc3_body(p0_ref, p1_ref, b2_ref, out_ref):
    z = p0_ref[...] + p1_ref[...] + b2_ref[...]
    col = lax.broadcasted_iota(i32, z.shape, 1)
    mask = col < C
    zm = jnp.where(mask, z, -1e30)
    m = jnp.max(zm, axis=1, keepdims=True)
    ex = jnp.where(mask, jnp.exp(z - m), 0.0)
    s = jnp.sum(ex, axis=1, keepdims=True)
    out_ref[...] = z - m - jnp.log(s)


def _tc3(p0, p1, b2r):
    bn = 1000
    return pl.pallas_call(
        _tc3_body,
        grid=(N // bn,),
        in_specs=[
            pl.BlockSpec((bn, CP), lambda i: (i, 0)),
            pl.BlockSpec((bn, CP), lambda i: (i, 0)),
            pl.BlockSpec((1, CP), lambda i: (0, 0)),
        ],
        out_specs=pl.BlockSpec((bn, CP), lambda i: (i, 0)),
        out_shape=jax.ShapeDtypeStruct((N, CP), f32),
    )(p0, p1, b2r)


def _sc1_body(elc, erc, h1f, src2, dst2, z16, z32, outg,
              elv, erv, sidx, didx, eebuf, scatbuf, dbuf, hbuf, msgbuf,
              dtab, outs):
    cid = lax.axis_index("c")
    sid = lax.axis_index("s")
    io = _iota16()
    m4 = io < 4

    pltpu.sync_copy(elc.at[cid], elv)
    pltpu.sync_copy(erc.at[cid], erv)
    nz0 = sid * 632
    pltpu.sync_copy(z16, dtab.at[pl.ds(nz0, 632)])
    pltpu.sync_copy(z32, outs.at[pl.ds(nz0, 632)])
    plsc.subcore_barrier()

    r0 = sid * (ROWS // NT)

    def group_a(g, _):
        rg = r0 + g * 8
        pltpu.sync_copy(src2.at[pl.ds(rg, 8)], sidx)
        pltpu.sync_copy(dst2.at[pl.ds(rg, 8)], didx)

        def row_a(m, _):
            for h in range(4):
                for s in range(8):
                    sv = sidx[m, pl.ds(s * 16, 16)]
                    dv = didx[m, pl.ds(s * 16, 16)]
                    a = plsc.load_gather(elv, [sv + h * N])
                    b = plsc.load_gather(erv, [dv + h * N])
                    e = a + b
                    e = jnp.maximum(e, 0.2 * e)
                    eebuf[pl.ds(m * 512 + h * 128 + s * 16, 16)] = jnp.exp(e)
            base = m * 512
            for e in range(128):
                idx = jnp.where(m4, base + io * 128 + e, 0)
                erow = plsc.load_gather(eebuf, [idx])
                scatbuf[e] = jnp.where(m4, erow, 0.0)
            pltpu.sync_copy(scatbuf, dtab.at[didx.at[m]], add=True)
            return 0

        lax.fori_loop(0, 8, row_a, 0)
        return 0

    lax.fori_loop(0, (ROWS // NT) // 8, group_a, 0)
    plsc.subcore_barrier()

    coff = cid * N

    def group_c(g, _):
        rg = r0 + g * 8
        pltpu.sync_copy(src2.at[pl.ds(rg, 8)], sidx)
        pltpu.sync_copy(dst2.at[pl.ds(rg, 8)], didx)

        def row_c(m, _):
            for h in range(4):
                for s in range(8):
                    sv = sidx[m, pl.ds(s * 16, 16)]
                    dv = didx[m, pl.ds(s * 16, 16)]
                    a = plsc.load_gather(elv, [sv + h * N])
                    b = plsc.load_gather(erv, [dv + h * N])
                    e = a + b
                    e = jnp.maximum(e, 0.2 * e)
                    eebuf[pl.ds(m * 512 + h * 128 + s * 16, 16)] = jnp.exp(e)
            for s in range(8):
                sidx[m, pl.ds(s * 16, 16)] = sidx[m, pl.ds(s * 16, 16)] + coff
            pltpu.sync_copy(h1f.at[sidx.at[m]], hbuf)
            pltpu.sync_copy(dtab.at[didx.at[m]], dbuf)
            base = m * 512
            for e in range(128):
                idx = jnp.where(m4, base + io * 128 + e, 0)
                erow = plsc.load_gather(eebuf, [idx])
                drow = dbuf[e]
                alpha = erow / (drow + 1e-9)
                for j in range(2):
                    pat = jnp.where(io < 8, 2 * j, 2 * j + 1)
                    aexp = _vperm(alpha, pat)
                    hv = hbuf[e, pl.ds(j * 16, 16)]
                    msgbuf[e, pl.ds(j * 16, 16)] = hv * aexp
            pltpu.sync_copy(msgbuf, outs.at[didx.at[m]], add=True)
            return 0

        lax.fori_loop(0, 8, row_c, 0)
        return 0

    lax.fori_loop(0, (ROWS // NT) // 8, group_c, 0)
    plsc.subcore_barrier()
    pltpu.sync_copy(outs.at[pl.ds(nz0, 632)], outg.at[cid, pl.ds(nz0, 632)])


def _sc1(elc, erc, h1f, src2, dst2, z16, z32):
    mesh = plsc.VectorSubcoreMesh(core_axis_name="c", subcore_axis_name="s")
    return pl.kernel(
        _sc1_body,
        out_type=jax.ShapeDtypeStruct((2, NP, 32), f32),
        mesh=mesh,
        compiler_params=pltpu.CompilerParams(needs_layout_passes=False, use_tc_tiling_on_sc=False),
        scratch_types=[
            pltpu.VMEM((4 * N,), f32),
            pltpu.VMEM((4 * N,), f32),
            pltpu.VMEM((8, 128), i32),
            pltpu.VMEM((8, 128), i32),
            pltpu.VMEM((8 * 512,), f32),
            pltpu.VMEM((128, 16), f32),
            pltpu.VMEM((128, 16), f32),
            pltpu.VMEM((128, 32), f32),
            pltpu.VMEM((128, 32), f32),
            pltpu.VMEM_SHARED((NP, 16), f32),
            pltpu.VMEM_SHARED((NP, 32), f32),
        ],
    )(elc, erc, h1f, src2, dst2, z16, z32)


def _sc2_body(el2, er2, h2g, src2, dst2, z16, z48, outg,
              elv, erv, sidx, didx, eebuf, scatbuf, dbuf, hbuf, msgbuf,
              ees, dtab, outs):
    cid = lax.axis_index("c")
    sid = lax.axis_index("s")
    io = _iota16()
    m1 = io < 1
    zvec = jnp.zeros((16,), i32)

    pltpu.sync_copy(el2, elv)
    pltpu.sync_copy(er2, erv)
    nz0 = sid * 632
    pltpu.sync_copy(z16, dtab.at[pl.ds(nz0, 632)])
    pltpu.sync_copy(z48, outs.at[pl.ds(nz0, 632)])
    plsc.subcore_barrier()

    r0 = sid * (ROWS // NT)
    half = ROWS // 2
    save = (sid * (ROWS // NT)) // half == cid

    def group_a(g, _):
        rg = r0 + g * 8
        pltpu.sync_copy(src2.at[pl.ds(rg, 8)], sidx)
        pltpu.sync_copy(dst2.at[pl.ds(rg, 8)], didx)

        def row_a(m, _):
            for s in range(8):
                sv = sidx[m, pl.ds(s * 16, 16)]
                dv = didx[m, pl.ds(s * 16, 16)]
                a = plsc.load_gather(elv, [sv])
                b = plsc.load_gather(erv, [dv])
                e = a + b
                e = jnp.maximum(e, 0.2 * e)
                ee = jnp.exp(e)
                eebuf[pl.ds(m * 128 + s * 16, 16)] = ee
                for l in range(16):
                    asp = _vperm(ee, jnp.full((16,), l, i32))
                    scatbuf[s * 16 + l] = jnp.where(m1, asp, 0.0)
            pltpu.sync_copy(scatbuf, dtab.at[didx.at[m]], add=True)
            return 0

        lax.fori_loop(0, 8, row_a, 0)

        @pl.when(save)
        def _():
            pltpu.sync_copy(eebuf, ees.at[pl.ds((rg - cid * half) * 128, 1024)])

        return 0

    lax.fori_loop(0, (ROWS // NT) // 8, group_a, 0)
    plsc.subcore_barrier()

    rpw = half // NT
    rc0 = sid * rpw

    def group_c(g, _):
        rl = rc0 + g * 8
        rg = cid * half + rl
        pltpu.sync_copy(src2.at[pl.ds(rg, 8)], sidx)
        pltpu.sync_copy(dst2.at[pl.ds(rg, 8)], didx)
        pltpu.sync_copy(ees.at[pl.ds(rl * 128, 1024)], eebuf)

        def row_c(m, _):
            pltpu.sync_copy(h2g.at[sidx.at[m]], hbuf)
            pltpu.sync_copy(dtab.at[didx.at[m]], dbuf)
            for s in range(8):
                dcol = plsc.load_gather(dbuf, [io + s * 16, zvec])
                al = eebuf[pl.ds(m * 128 + s * 16, 16)] / (dcol + 1e-9)
                for l in range(16):
                    asp = _vperm(al, jnp.full((16,), l, i32))
                    for j in range(3):
                        hv = hbuf[s * 16 + l, pl.ds(j * 16, 16)]
                        msgbuf[s * 16 + l, pl.ds(j * 16, 16)] = hv * asp
            pltpu.sync_copy(msgbuf, outs.at[didx.at[m]], add=True)
            return 0

        lax.fori_loop(0, 8, row_c, 0)
        return 0

    lax.fori_loop(0, rpw // 8, group_c, 0)
    plsc.subcore_barrier()
    pltpu.sync_copy(outs.at[pl.ds(nz0, 632)], outg.at[cid, pl.ds(nz0, 632)])


def _sc2(el2, er2, h2g, src2, dst2, z16, z48):
    mesh = plsc.VectorSubcoreMesh(core_axis_name="c", subcore_axis_name="s")
    return pl.kernel(
        _sc2_body,
        out_type=jax.ShapeDtypeStruct((2, NP, CP), f32),
        mesh=mesh,
        compiler_params=pltpu.CompilerParams(needs_layout_passes=False, use_tc_tiling_on_sc=False),
        scratch_types=[
            pltpu.VMEM((N,), f32),
            pltpu.VMEM((N,), f32),
            pltpu.VMEM((8, 128), i32),
            pltpu.VMEM((8, 128), i32),
            pltpu.VMEM((8 * 128,), f32),
            pltpu.VMEM((128, 16), f32),
            pltpu.VMEM((128, 16), f32),
            pltpu.VMEM((128, CP), f32),
            pltpu.VMEM((128, CP), f32),
            pltpu.VMEM_SHARED((ROWS // 2 * 128,), f32),
            pltpu.VMEM_SHARED((NP, 16), f32),
            pltpu.VMEM_SHARED((NP, CP), f32),
        ],
    )(el2, er2, h2g, src2, dst2, z16, z48)


@jax.jit
def kernel(inputs, edge_index, W1, b1, al1, ar1, W2, b2, al2, ar2):
    src = edge_index[0].astype(i32)
    dst = edge_index[1].astype(i32)

    eye = jnp.repeat(jnp.eye(H1, dtype=f32), D1, axis=0)
    albk = eye * al1.reshape(H1 * D1, 1)
    arbk = eye * ar1.reshape(H1 * D1, 1)

    h1g, el, er = _tc1(inputs, W1, albk, arbk)

    npad = EP - E
    srcp = jnp.concatenate([src, jnp.zeros((npad,), i32)]).reshape(ROWS, 128)
    dstp = jnp.concatenate([dst, jnp.full((npad,), N, i32)]).reshape(ROWS, 128)

    elc = el.T.reshape(2, 4 * N)
    erc = er.T.reshape(2, 4 * N)
    h1f = h1g.reshape(2 * N, 32)
    z16 = jnp.zeros((632, 16), f32)
    z32 = jnp.zeros((632, 32), f32)
    z48 = jnp.zeros((632, CP), f32)

    o1g = _sc1(elc, erc, h1f, srcp, dstp, z16, z32)
    o1 = jnp.concatenate([o1g[0, :N], o1g[1, :N]], axis=1)

    w2p = jnp.pad(W2, ((0, 0), (0, CP - C)))
    a2l = jnp.pad(al2.reshape(C, 1), ((0, CP - C), (0, 0)))
    a2r = jnp.pad(ar2.reshape(C, 1), ((0, CP - C), (0, 0)))
    h2g, el2, er2 = _tc2(o1, w2p, b1.reshape(1, 64), a2l, a2r)

    o2g = _sc2(el2.reshape(N), er2.reshape(N), h2g, srcp, dstp, z16, z48)

    b2r = jnp.pad(b2, (0, CP - C)).reshape(1, CP)
    out = _tc3(o2g[0, :N], o2g[1, :N], b2r)
    return out[:, :C]


# speedup vs baseline: 1.2510x; 1.0029x over previous
import functools

import jax
import jax.numpy as jnp
from jax import lax
from jax.experimental import pallas as pl
from jax.experimental.pallas import tpu as pltpu
from jax.experimental.pallas import tpu_sc as plsc

N = 10000
E = 320000
F = 128
H1, D1 = 8, 8
C = 40
CP = 48
NP = 10112
EP = 327680
ROWS = EP // 128
NT = 16
NC = 2

f32 = jnp.float32
i32 = jnp.int32


def _iota16():
    return lax.iota(i32, 16)


def _vperm(x, idx):
    dnums = lax.GatherDimensionNumbers(
        offset_dims=(), collapsed_slice_dims=(0,), start_index_map=(0,))
    return lax.gather(x, idx[:, None], dnums, (1,),
                      mode=lax.GatherScatterMode.PROMISE_IN_BOUNDS)


def _tc1_body(x_ref, w_ref, al_ref, ar_ref, h_ref, el_ref, er_ref):
    h = jnp.dot(x_ref[...], w_ref[...], preferred_element_type=f32)
    h_ref[0] = h[:, :32]
    h_ref[1] = h[:, 32:]
    el_ref[...] = jnp.dot(h, al_ref[...], preferred_element_type=f32)
    er_ref[...] = jnp.dot(h, ar_ref[...], preferred_element_type=f32)


def _tc1(x, w1, albk, arbk):
    bn = 1000
    return pl.pallas_call(
        _tc1_body,
        grid=(N // bn,),
        in_specs=[
            pl.BlockSpec((bn, F), lambda i: (i, 0)),
            pl.BlockSpec((F, 64), lambda i: (0, 0)),
            pl.BlockSpec((64, 8), lambda i: (0, 0)),
            pl.BlockSpec((64, 8), lambda i: (0, 0)),
        ],
        out_specs=[
            pl.BlockSpec((2, bn, 32), lambda i: (0, i, 0)),
            pl.BlockSpec((bn, 8), lambda i: (i, 0)),
            pl.BlockSpec((bn, 8), lambda i: (i, 0)),
        ],
        out_shape=[
            jax.ShapeDtypeStruct((2, N, 32), f32),
            jax.ShapeDtypeStruct((N, 8), f32),
            jax.ShapeDtypeStruct((N, 8), f32),
        ],
    )(x, w1, albk, arbk)


def _tc2_body(o1_ref, w2_ref, b1_ref, a2l_ref, a2r_ref, h2_ref, el_ref, er_ref):
    v = o1_ref[...] + b1_ref[...]
    y = jnp.where(v > 0, v, jnp.exp(v) - 1.0)
    h2 = jnp.dot(y, w2_ref[...], preferred_element_type=f32)
    h2_ref[...] = h2
    el_ref[...] = jnp.dot(h2, a2l_ref[...], preferred_element_type=f32)
    er_ref[...] = jnp.dot(h2, a2r_ref[...], preferred_element_type=f32)


def _tc2(o1, w2p, b1r, a2l, a2r):
    bn = 1000
    return pl.pallas_call(
        _tc2_body,
        grid=(N // bn,),
        in_specs=[
            pl.BlockSpec((bn, 64), lambda i: (i, 0)),
            pl.BlockSpec((64, CP), lambda i: (0, 0)),
            pl.BlockSpec((1, 64), lambda i: (0, 0)),
            pl.BlockSpec((CP, 1), lambda i: (0, 0)),
            pl.BlockSpec((CP, 1), lambda i: (0, 0)),
        ],
        out_specs=[
            pl.BlockSpec((bn, CP), lambda i: (i, 0)),
            pl.BlockSpec((bn, 1), lambda i: (i, 0)),
            pl.BlockSpec((bn, 1), lambda i: (i, 0)),
        ],
        out_shape=[
            jax.ShapeDtypeStruct((N, CP), f32),
            jax.ShapeDtypeStruct((N, 1), f32),
            jax.ShapeDtypeStruct((N, 1), f32),
        ],
    )(o1, w2p, b1r, a2l, a2r)


def _tc3_body(p0_ref, p1_ref, b2_ref, out_ref):
    z = p0_ref[...] + p1_ref[...] + b2_ref[...]
    col = lax.broadcasted_iota(i32, z.shape, 1)
    mask = col < C
    zm = jnp.where(mask, z, -1e30)
    m = jnp.max(zm, axis=1, keepdims=True)
    ex = jnp.where(mask, jnp.exp(z - m), 0.0)
    s = jnp.sum(ex, axis=1, keepdims=True)
    out_ref[...] = z - m - jnp.log(s)


def _tc3(p0, p1, b2r):
    bn = 1000
    return pl.pallas_call(
        _tc3_body,
        grid=(N // bn,),
        in_specs=[
            pl.BlockSpec((bn, CP), lambda i: (i, 0)),
            pl.BlockSpec((bn, CP), lambda i: (i, 0)),
            pl.BlockSpec((1, CP), lambda i: (0, 0)),
        ],
        out_specs=pl.BlockSpec((bn, CP), lambda i: (i, 0)),
        out_shape=jax.ShapeDtypeStruct((N, CP), f32),
    )(p0, p1, b2r)


def _sc1_body(elc, erc, h1f, src2, dst2, z16, z32, outg,
              elv, erv, sidx, didx, eebuf, scatbuf, dbuf, hbuf, msgbuf,
              dtab, outs):
    cid = lax.axis_index("c")
    sid = lax.axis_index("s")
    io = _iota16()
    m4 = io < 4

    pltpu.sync_copy(elc.at[cid], elv)
    pltpu.sync_copy(erc.at[cid], erv)
    nz0 = sid * 632
    pltpu.sync_copy(z16, dtab.at[pl.ds(nz0, 632)])
    pltpu.sync_copy(z32, outs.at[pl.ds(nz0, 632)])
    plsc.subcore_barrier()

    r0 = sid * (ROWS // NT)

    def group_a(g, _):
        rg = r0 + g * 8
        pltpu.sync_copy(src2.at[pl.ds(rg, 8)], sidx)
        pltpu.sync_copy(dst2.at[pl.ds(rg, 8)], didx)

        def row_a(m, _):
            for h in range(4):
                for s in range(8):
                    sv = sidx[m, pl.ds(s * 16, 16)]
                    dv = didx[m, pl.ds(s * 16, 16)]
                    a = plsc.load_gather(elv, [sv + h * N])
                    b = plsc.load_gather(erv, [dv + h * N])
                    e = a + b
                    e = jnp.maximum(e, 0.2 * e)
                    eebuf[pl.ds(m * 512 + h * 128 + s * 16, 16)] = jnp.exp(e)
            base = m * 512
            for e in range(128):
                idx = jnp.where(m4, base + io * 128 + e, 0)
                erow = plsc.load_gather(eebuf, [idx])
                scatbuf[e] = jnp.where(m4, erow, 0.0)
            pltpu.sync_copy(scatbuf, dtab.at[didx.at[m]], add=True)
            return 0

        lax.fori_loop(0, 8, row_a, 0)
        return 0

    lax.fori_loop(0, (ROWS // NT) // 8, group_a, 0)
    plsc.subcore_barrier()

    def inv_c(c, _):
        b0 = nz0 + c * 8
        pltpu.sync_copy(dtab.at[pl.ds(b0, 8)], scatbuf.at[pl.ds(0, 8)])
        for rr in range(8):
            scatbuf[rr] = 1.0 / (scatbuf[rr] + 1e-9)
        pltpu.sync_copy(scatbuf.at[pl.ds(0, 8)], dtab.at[pl.ds(b0, 8)])
        return 0

    lax.fori_loop(0, 632 // 8, inv_c, 0)
    plsc.subcore_barrier()

    coff = cid * N

    def group_c(g, _):
        rg = r0 + g * 8
        pltpu.sync_copy(src2.at[pl.ds(rg, 8)], sidx)
        pltpu.sync_copy(dst2.at[pl.ds(rg, 8)], didx)

        def row_c(m, _):
            for h in range(4):
                for s in range(8):
                    sv = sidx[m, pl.ds(s * 16, 16)]
                    dv = didx[m, pl.ds(s * 16, 16)]
                    a = plsc.load_gather(elv, [sv + h * N])
                    b = plsc.load_gather(erv, [dv + h * N])
                    e = a + b
                    e = jnp.maximum(e, 0.2 * e)
                    eebuf[pl.ds(m * 512 + h * 128 + s * 16, 16)] = jnp.exp(e)
            for s in range(8):
                sidx[m, pl.ds(s * 16, 16)] = sidx[m, pl.ds(s * 16, 16)] + coff
            pltpu.sync_copy(h1f.at[sidx.at[m]], hbuf)
            pltpu.sync_copy(dtab.at[didx.at[m]], dbuf)
            base = m * 512
            for e in range(128):
                idx = jnp.where(m4, base + io * 128 + e, 0)
                erow = plsc.load_gather(eebuf, [idx])
                drow = dbuf[e]
                alpha = erow * drow
                for j in range(2):
                    pat = jnp.where(io < 8, 2 * j, 2 * j + 1)
                    aexp = _vperm(alpha, pat)
                    hv = hbuf[e, pl.ds(j * 16, 16)]
                    msgbuf[e, pl.ds(j * 16, 16)] = hv * aexp
            pltpu.sync_copy(msgbuf, outs.at[didx.at[m]], add=True)
            return 0

        lax.fori_loop(0, 8, row_c, 0)
        return 0

    lax.fori_loop(0, (ROWS // NT) // 8, group_c, 0)
    plsc.subcore_barrier()
    pltpu.sync_copy(outs.at[pl.ds(nz0, 632)], outg.at[cid, pl.ds(nz0, 632)])


def _sc1(elc, erc, h1f, src2, dst2, z16, z32):
    mesh = plsc.VectorSubcoreMesh(core_axis_name="c", subcore_axis_name="s")
    return pl.kernel(
        _sc1_body,
        out_type=jax.ShapeDtypeStruct((2, NP, 32), f32),
        mesh=mesh,
        compiler_params=pltpu.CompilerParams(needs_layout_passes=False, use_tc_tiling_on_sc=False),
        scratch_types=[
            pltpu.VMEM((4 * N,), f32),
            pltpu.VMEM((4 * N,), f32),
            pltpu.VMEM((8, 128), i32),
            pltpu.VMEM((8, 128), i32),
            pltpu.VMEM((8 * 512,), f32),
            pltpu.VMEM((128, 16), f32),
            pltpu.VMEM((128, 16), f32),
            pltpu.VMEM((128, 32), f32),
            pltpu.VMEM((128, 32), f32),
            pltpu.VMEM_SHARED((NP, 16), f32),
            pltpu.VMEM_SHARED((NP, 32), f32),
        ],
    )(elc, erc, h1f, src2, dst2, z16, z32)


def _sc2_body(el2, er2, h2g, src2, dst2, z16, z48, outg,
              elv, erv, sidx, didx, eebuf, scatbuf, dbuf, hbuf, msgbuf,
              ees, dtab, outs):
    cid = lax.axis_index("c")
    sid = lax.axis_index("s")
    io = _iota16()
    m1 = io < 1
    zvec = jnp.zeros((16,), i32)

    pltpu.sync_copy(el2, elv)
    pltpu.sync_copy(er2, erv)
    nz0 = sid * 632
    pltpu.sync_copy(z16, dtab.at[pl.ds(nz0, 632)])
    pltpu.sync_copy(z48, outs.at[pl.ds(nz0, 632)])
    plsc.subcore_barrier()

    r0 = sid * (ROWS // NT)
    half = ROWS // 2
    save = (sid * (ROWS // NT)) // half == cid

    def group_a(g, _):
        rg = r0 + g * 8
        pltpu.sync_copy(src2.at[pl.ds(rg, 8)], sidx)
        pltpu.sync_copy(dst2.at[pl.ds(rg, 8)], didx)

        def row_a(m, _):
            for s in range(8):
                sv = sidx[m, pl.ds(s * 16, 16)]
                dv = didx[m, pl.ds(s * 16, 16)]
                a = plsc.load_gather(elv, [sv])
                b = plsc.load_gather(erv, [dv])
                e = a + b
                e = jnp.maximum(e, 0.2 * e)
                ee = jnp.exp(e)
                eebuf[pl.ds(m * 128 + s * 16, 16)] = ee
                for l in range(16):
                    asp = _vperm(ee, jnp.full((16,), l, i32))
                    scatbuf[s * 16 + l] = jnp.where(m1, asp, 0.0)
            pltpu.sync_copy(scatbuf, dtab.at[didx.at[m]], add=True)
            return 0

        lax.fori_loop(0, 8, row_a, 0)

        @pl.when(save)
        def _():
            pltpu.sync_copy(eebuf, ees.at[pl.ds((rg - cid * half) * 128, 1024)])

        return 0

    lax.fori_loop(0, (ROWS // NT) // 8, group_a, 0)
    plsc.subcore_barrier()

    def inv_c(c, _):
        b0 = nz0 + c * 8
        pltpu.sync_copy(dtab.at[pl.ds(b0, 8)], scatbuf.at[pl.ds(0, 8)])
        for rr in range(8):
            scatbuf[rr] = 1.0 / (scatbuf[rr] + 1e-9)
        pltpu.sync_copy(scatbuf.at[pl.ds(0, 8)], dtab.at[pl.ds(b0, 8)])
        return 0

    lax.fori_loop(0, 632 // 8, inv_c, 0)
    plsc.subcore_barrier()

    rpw = half // NT
    rc0 = sid * rpw

    def group_c(g, _):
        rl = rc0 + g * 8
        rg = cid * half + rl
        pltpu.sync_copy(src2.at[pl.ds(rg, 8)], sidx)
        pltpu.sync_copy(dst2.at[pl.ds(rg, 8)], didx)
        pltpu.sync_copy(ees.at[pl.ds(rl * 128, 1024)], eebuf)

        def row_c(m, _):
            pltpu.sync_copy(h2g.at[sidx.at[m]], hbuf)
            pltpu.sync_copy(dtab.at[didx.at[m]], dbuf)
            for s in range(8):
                dcol = plsc.load_gather(dbuf, [io + s * 16, zvec])
                al = eebuf[pl.ds(m * 128 + s * 16, 16)] * dcol
                for l in range(16):
                    asp = _vperm(al, jnp.full((16,), l, i32))
                    for j in range(3):
                        hv = hbuf[s * 16 + l, pl.ds(j * 16, 16)]
                        msgbuf[s * 16 + l, pl.ds(j * 16, 16)] = hv * asp
            pltpu.sync_copy(msgbuf, outs.at[didx.at[m]], add=True)
            return 0

        lax.fori_loop(0, 8, row_c, 0)
        return 0

    lax.fori_loop(0, rpw // 8, group_c, 0)
    plsc.subcore_barrier()
    pltpu.sync_copy(outs.at[pl.ds(nz0, 632)], outg.at[cid, pl.ds(nz0, 632)])


def _sc2(el2, er2, h2g, src2, dst2, z16, z48):
    mesh = plsc.VectorSubcoreMesh(core_axis_name="c", subcore_axis_name="s")
    return pl.kernel(
        _sc2_body,
        out_type=jax.ShapeDtypeStruct((2, NP, CP), f32),
        mesh=mesh,
        compiler_params=pltpu.CompilerParams(needs_layout_passes=False, use_tc_tiling_on_sc=False),
        scratch_types=[
            pltpu.VMEM((N,), f32),
            pltpu.VMEM((N,), f32),
            pltpu.VMEM((8, 128), i32),
            pltpu.VMEM((8, 128), i32),
            pltpu.VMEM((8 * 128,), f32),
            pltpu.VMEM((128, 16), f32),
            pltpu.VMEM((128, 16), f32),
            pltpu.VMEM((128, CP), f32),
            pltpu.VMEM((128, CP), f32),
            pltpu.VMEM_SHARED((ROWS // 2 * 128,), f32),
            pltpu.VMEM_SHARED((NP, 16), f32),
            pltpu.VMEM_SHARED((NP, CP), f32),
        ],
    )(el2, er2, h2g, src2, dst2, z16, z48)


@jax.jit
def kernel(inputs, edge_index, W1, b1, al1, ar1, W2, b2, al2, ar2):
    src = edge_index[0].astype(i32)
    dst = edge_index[1].astype(i32)

    eye = jnp.repeat(jnp.eye(H1, dtype=f32), D1, axis=0)
    albk = eye * al1.reshape(H1 * D1, 1)
    arbk = eye * ar1.reshape(H1 * D1, 1)

    h1g, el, er = _tc1(inputs, W1, albk, arbk)

    npad = EP - E
    srcp = jnp.concatenate([src, jnp.zeros((npad,), i32)]).reshape(ROWS, 128)
    dstp = jnp.concatenate([dst, jnp.full((npad,), N, i32)]).reshape(ROWS, 128)

    elc = el.T.reshape(2, 4 * N)
    erc = er.T.reshape(2, 4 * N)
    h1f = h1g.reshape(2 * N, 32)
    z16 = jnp.zeros((632, 16), f32)
    z32 = jnp.zeros((632, 32), f32)
    z48 = jnp.zeros((632, CP), f32)

    o1g = _sc1(elc, erc, h1f, srcp, dstp, z16, z32)
    o1 = jnp.concatenate([o1g[0, :N], o1g[1, :N]], axis=1)

    w2p = jnp.pad(W2, ((0, 0), (0, CP - C)))
    a2l = jnp.pad(al2.reshape(C, 1), ((0, CP - C), (0, 0)))
    a2r = jnp.pad(ar2.reshape(C, 1), ((0, CP - C), (0, 0)))
    h2g, el2, er2 = _tc2(o1, w2p, b1.reshape(1, 64), a2l, a2r)

    o2g = _sc2(el2.reshape(N), er2.reshape(N), h2g, srcp, dstp, z16, z48)

    b2r = jnp.pad(b2, (0, CP - C)).reshape(1, CP)
    out = _tc3(o2g[0, :N], o2g[1, :N], b2r)
    return out[:, :C]
